# Initial kernel scaffold; baseline (speedup 1.0000x reference)
import jax
import jax.numpy as jnp
from jax.experimental import pallas as pl


def kernel(x, ygen_id, ygen, params):
    raise NotImplementedError("write your pallas kernel here")


# speedup vs baseline: 4.2409x; 4.2409x over previous
import functools

import jax
import jax.numpy as jnp
from jax.experimental import pallas as pl

_K = 16
_R = 256


def _lrelu(t):
    return jnp.where(t >= 0, t, 0.01 * t)


def _dot(a, b):
    return jnp.dot(a, b, preferred_element_type=jnp.float32)


def _encoder_body(x_ref, w1, b1, w2, b2, w3, b3, ws, bs, wh, bh,
                  x1_ref, s_ref, hp_ref):
    x = x_ref[...]
    h = _lrelu(_dot(x, w1[...]) + b1[...])
    h = _lrelu(_dot(h, w2[...]) + b2[...])
    x1 = _dot(h, w3[...]) + b3[...]
    x1_ref[...] = x1
    s_ref[...] = _dot(x1, ws[...]) + bs[...]
    hp_ref[...] = _dot(x1, wh[...]) + bh[...]


def _knn_body(n_real, s_ref, sT_ref, hs_ref, x1_ref, wo, bo,
              xg_ref, idx_ref, ew_ref):
    tile = pl.program_id(0)
    R = s_ref.shape[0]
    npad = sT_ref.shape[1]
    s_r = s_ref[...]
    sT = sT_ref[...]
    sq_r = jnp.sum(s_r * s_r, axis=1, keepdims=True)
    sq_c = jnp.sum(sT * sT, axis=0, keepdims=True)
    d2 = sq_r + sq_c - 2.0 * _dot(s_r, sT[:4, :])
    col = jax.lax.broadcasted_iota(jnp.int32, (R, npad), 1)
    row = tile * R + jax.lax.broadcasted_iota(jnp.int32, (R, npad), 0)
    big = jnp.float32(1e30)
    d2 = jnp.where((col >= n_real) | (col == row), big, d2)

    hs = hs_ref[...]
    mean_acc = jnp.zeros((R, 22), jnp.float32)
    max_acc = jnp.full((R, 22), -big, jnp.float32)
    idx_cols = []
    ew_cols = []
    for _ in range(_K):
        m = jnp.min(d2, axis=1, keepdims=True)
        sel = d2 == m
        i_sel = jnp.min(jnp.where(sel, col, npad), axis=1, keepdims=True)
        onehot = (col == i_sel).astype(jnp.float32)
        g = _dot(onehot, hs)
        diff = g[:, 22:26] - s_r
        d2e = jnp.sum(diff * diff, axis=1, keepdims=True)
        w_t = jnp.exp(-10.0 * d2e)
        msg = g[:, :22] * w_t
        mean_acc = mean_acc + msg
        max_acc = jnp.maximum(max_acc, msg)
        idx_cols.append(i_sel)
        ew_cols.append(w_t)
        d2 = jnp.where(col == i_sel, big, d2)
    mean_acc = mean_acc * (1.0 / _K)

    out = jnp.concatenate([mean_acc, max_acc, x1_ref[...]], axis=1)
    xg_ref[...] = _lrelu(_dot(out, wo[...]) + bo[...])
    idx_ref[...] = jnp.concatenate(idx_cols, axis=1).astype(jnp.int32)
    ew_ref[...] = jnp.concatenate(ew_cols, axis=1)


def _head_body(idx_ref, ew_ref, xgf_ref, xg_ref, wrel, brel, wroot,
               w21, b21, w22, b22, w23, b23, w31, b31, w32, b32, w33, b33,
               ids_ref, p4_ref):
    R = idx_ref.shape[0]
    npad = xgf_ref.shape[0]
    idx = idx_ref[...]
    ew = ew_ref[...]
    col = jax.lax.broadcasted_iota(jnp.int32, (R, npad), 1)
    w = jnp.zeros((R, npad), jnp.float32)
    for t in range(_K):
        w = w + jnp.where(col == idx[:, t:t + 1], ew[:, t:t + 1], 0.0)
    agg2 = _dot(w, xgf_ref[...])
    xg = xg_ref[...]
    xc = _lrelu(agg2 @ wrel[...] + brel[...] + _dot(xg, wroot[...]))
    h2 = _lrelu(_dot(xc, w21[...]) + b21[...])
    h2 = _lrelu(_dot(h2, w22[...]) + b22[...])
    ids = _dot(h2, w23[...]) + b23[...]
    z = jnp.concatenate([xc, ids], axis=1)
    h3 = _lrelu(_dot(z, w31[...]) + b31[...])
    h3 = _lrelu(_dot(h3, w32[...]) + b32[...])
    p4 = _dot(h3, w33[...]) + b33[...]
    ids_ref[...] = ids
    p4_ref[...] = p4


def _full(shape):
    nd = len(shape)
    return pl.BlockSpec(shape, lambda i: (0,) * nd)


def _tiled(c):
    return pl.BlockSpec((_R, c), lambda i: (i, 0))


def kernel(x, ygen_id, ygen, params):
    p = params
    n = x.shape[0]
    npad = -(-n // _R) * _R
    grid = npad // _R
    f32 = jnp.float32

    xp = jnp.zeros((npad, 12), f32).at[:n, :].set(x)

    def b2(name):
        return p[name + '_b'].reshape(1, -1)

    x1, s, hp = pl.pallas_call(
        _encoder_body,
        grid=(grid,),
        in_specs=[_tiled(12)] + [_full(p[k].shape if k.endswith('_W')
                                       else (1, p[k[:-2] + '_b'].shape[0]))
                                 for k in ['nn1_1_W', 'nn1_1_b', 'nn1_2_W',
                                           'nn1_2_b', 'nn1_3_W', 'nn1_3_b',
                                           'gn_s_W', 'gn_s_b', 'gn_h_W',
                                           'gn_h_b']],
        out_specs=[_tiled(12), _tiled(4), _tiled(22)],
        out_shape=[jax.ShapeDtypeStruct((npad, 12), f32),
                   jax.ShapeDtypeStruct((npad, 4), f32),
                   jax.ShapeDtypeStruct((npad, 22), f32)],
    )(xp, p['nn1_1_W'], b2('nn1_1'), p['nn1_2_W'], b2('nn1_2'),
      p['nn1_3_W'], b2('nn1_3'), p['gn_s_W'], b2('gn_s'),
      p['gn_h_W'], b2('gn_h'))

    sT = jnp.zeros((8, npad), f32).at[:4, :].set(s.T)
    hs = jnp.concatenate([hp, s], axis=1)

    xg, idx, ew = pl.pallas_call(
        functools.partial(_knn_body, n),
        grid=(grid,),
        in_specs=[_tiled(4), _full((8, npad)), _full((npad, 26)), _tiled(12),
                  _full((56, 32)), _full((1, 32))],
        out_specs=[_tiled(32), _tiled(_K), _tiled(_K)],
        out_shape=[jax.ShapeDtypeStruct((npad, 32), f32),
                   jax.ShapeDtypeStruct((npad, _K), jnp.int32),
                   jax.ShapeDtypeStruct((npad, _K), f32)],
    )(s, sT, hs, x1, p['gn_o_W'], b2('gn_o'))

    ids, p4 = pl.pallas_call(
        _head_body,
        grid=(grid,),
        in_specs=[_tiled(_K), _tiled(_K), _full((npad, 32)), _tiled(32),
                  _full((32, 32)), _full((1, 32)), _full((32, 32)),
                  _full((32, 125)), _full((1, 125)),
                  _full((125, 125)), _full((1, 125)),
                  _full((125, 6)), _full((1, 6)),
                  _full((38, 125)), _full((1, 125)),
                  _full((125, 125)), _full((1, 125)),
                  _full((125, 6)), _full((1, 6))],
        out_specs=[_tiled(6), _tiled(6)],
        out_shape=[jax.ShapeDtypeStruct((npad, 6), f32),
                   jax.ShapeDtypeStruct((npad, 6), f32)],
    )(idx, ew, xg, xg, p['gc_rel_W'], b2('gc_rel'), p['gc_root_W'],
      p['nn2_1_W'], b2('nn2_1'), p['nn2_2_W'], b2('nn2_2'),
      p['nn2_3_W'], b2('nn2_3'), p['nn3_1_W'], b2('nn3_1'),
      p['nn3_2_W'], b2('nn3_2'), p['nn3_3_W'], b2('nn3_3'))

    return (ids[:n], p4[:n], ygen_id, ygen)


# speedup vs baseline: 4.2951x; 1.0128x over previous
import functools

import jax
import jax.numpy as jnp
from jax.experimental import pallas as pl

_K = 16
_R = 256


def _lrelu(t):
    return jnp.where(t >= 0, t, 0.01 * t)


def _dot(a, b):
    return jnp.dot(a, b, preferred_element_type=jnp.float32)


def _encoder_body(x_ref, w1, b1, w2, b2, w3, b3, ws, bs, wh, bh,
                  x1_ref, s_ref, hp_ref):
    x = x_ref[...]
    h = _lrelu(_dot(x, w1[...]) + b1[...])
    h = _lrelu(_dot(h, w2[...]) + b2[...])
    x1 = _dot(h, w3[...]) + b3[...]
    x1_ref[...] = x1
    s_ref[...] = _dot(x1, ws[...]) + bs[...]
    hp_ref[...] = _dot(x1, wh[...]) + bh[...]


def _knn_body(n_real, s_ref, sT_ref, hs_ref, x1_ref, wo, bo,
              xg_ref, idx_ref, ew_ref):
    tile = pl.program_id(0)
    R = s_ref.shape[0]
    npad = sT_ref.shape[1]
    s_r = s_ref[...]
    sT = sT_ref[...]
    sq_r = jnp.sum(s_r * s_r, axis=1, keepdims=True)
    sq_c = jnp.sum(sT * sT, axis=0, keepdims=True)
    d2 = sq_r + sq_c - 2.0 * _dot(s_r, sT[:4, :])
    col = jax.lax.broadcasted_iota(jnp.int32, (R, npad), 1)
    row = tile * R + jax.lax.broadcasted_iota(jnp.int32, (R, npad), 0)
    big = jnp.float32(1e30)
    d2 = jnp.where((col >= n_real) | (col == row), big, d2)

    hs = hs_ref[...]
    mean_acc = jnp.zeros((R, 22), jnp.float32)
    max_acc = jnp.full((R, 22), -big, jnp.float32)
    idx_cols = []
    ew_cols = []
    for _ in range(_K):
        i_sel = jnp.argmin(d2, axis=1, keepdims=True).astype(jnp.int32)
        sel = col == i_sel
        g = _dot(sel.astype(jnp.float32), hs)
        diff = g[:, 22:26] - s_r
        d2e = jnp.sum(diff * diff, axis=1, keepdims=True)
        w_t = jnp.exp(-10.0 * d2e)
        msg = g[:, :22] * w_t
        mean_acc = mean_acc + msg
        max_acc = jnp.maximum(max_acc, msg)
        idx_cols.append(i_sel)
        ew_cols.append(w_t)
        d2 = jnp.where(sel, big, d2)
    mean_acc = mean_acc * (1.0 / _K)

    out = jnp.concatenate([mean_acc, max_acc, x1_ref[...]], axis=1)
    xg_ref[...] = _lrelu(_dot(out, wo[...]) + bo[...])
    idx_ref[...] = jnp.concatenate(idx_cols, axis=1).astype(jnp.int32)
    ew_ref[...] = jnp.concatenate(ew_cols, axis=1)


def _head_body(idx_ref, ew_ref, xgf_ref, xg_ref, wrel, brel, wroot,
               w21, b21, w22, b22, w23, b23, w31, b31, w32, b32, w33, b33,
               ids_ref, p4_ref):
    R = idx_ref.shape[0]
    npad = xgf_ref.shape[0]
    idx = idx_ref[...]
    ew = ew_ref[...]
    col = jax.lax.broadcasted_iota(jnp.int32, (R, npad), 1)
    w = jnp.zeros((R, npad), jnp.float32)
    for t in range(_K):
        w = w + jnp.where(col == idx[:, t:t + 1], ew[:, t:t + 1], 0.0)
    agg2 = _dot(w, xgf_ref[...])
    xg = xg_ref[...]
    xc = _lrelu(agg2 @ wrel[...] + brel[...] + _dot(xg, wroot[...]))
    h2 = _lrelu(_dot(xc, w21[...]) + b21[...])
    h2 = _lrelu(_dot(h2, w22[...]) + b22[...])
    ids = _dot(h2, w23[...]) + b23[...]
    z = jnp.concatenate([xc, ids], axis=1)
    h3 = _lrelu(_dot(z, w31[...]) + b31[...])
    h3 = _lrelu(_dot(h3, w32[...]) + b32[...])
    p4 = _dot(h3, w33[...]) + b33[...]
    ids_ref[...] = ids
    p4_ref[...] = p4


def _full(shape):
    nd = len(shape)
    return pl.BlockSpec(shape, lambda i: (0,) * nd)


def _tiled(c):
    return pl.BlockSpec((_R, c), lambda i: (i, 0))


def kernel(x, ygen_id, ygen, params):
    p = params
    n = x.shape[0]
    npad = -(-n // _R) * _R
    grid = npad // _R
    f32 = jnp.float32

    xp = jnp.zeros((npad, 12), f32).at[:n, :].set(x)

    def b2(name):
        return p[name + '_b'].reshape(1, -1)

    x1, s, hp = pl.pallas_call(
        _encoder_body,
        grid=(grid,),
        in_specs=[_tiled(12)] + [_full(p[k].shape if k.endswith('_W')
                                       else (1, p[k[:-2] + '_b'].shape[0]))
                                 for k in ['nn1_1_W', 'nn1_1_b', 'nn1_2_W',
                                           'nn1_2_b', 'nn1_3_W', 'nn1_3_b',
                                           'gn_s_W', 'gn_s_b', 'gn_h_W',
                                           'gn_h_b']],
        out_specs=[_tiled(12), _tiled(4), _tiled(22)],
        out_shape=[jax.ShapeDtypeStruct((npad, 12), f32),
                   jax.ShapeDtypeStruct((npad, 4), f32),
                   jax.ShapeDtypeStruct((npad, 22), f32)],
    )(xp, p['nn1_1_W'], b2('nn1_1'), p['nn1_2_W'], b2('nn1_2'),
      p['nn1_3_W'], b2('nn1_3'), p['gn_s_W'], b2('gn_s'),
      p['gn_h_W'], b2('gn_h'))

    sT = jnp.zeros((8, npad), f32).at[:4, :].set(s.T)
    hs = jnp.concatenate([hp, s], axis=1)

    xg, idx, ew = pl.pallas_call(
        functools.partial(_knn_body, n),
        grid=(grid,),
        in_specs=[_tiled(4), _full((8, npad)), _full((npad, 26)), _tiled(12),
                  _full((56, 32)), _full((1, 32))],
        out_specs=[_tiled(32), _tiled(_K), _tiled(_K)],
        out_shape=[jax.ShapeDtypeStruct((npad, 32), f32),
                   jax.ShapeDtypeStruct((npad, _K), jnp.int32),
                   jax.ShapeDtypeStruct((npad, _K), f32)],
    )(s, sT, hs, x1, p['gn_o_W'], b2('gn_o'))

    ids, p4 = pl.pallas_call(
        _head_body,
        grid=(grid,),
        in_specs=[_tiled(_K), _tiled(_K), _full((npad, 32)), _tiled(32),
                  _full((32, 32)), _full((1, 32)), _full((32, 32)),
                  _full((32, 125)), _full((1, 125)),
                  _full((125, 125)), _full((1, 125)),
                  _full((125, 6)), _full((1, 6)),
                  _full((38, 125)), _full((1, 125)),
                  _full((125, 125)), _full((1, 125)),
                  _full((125, 6)), _full((1, 6))],
        out_specs=[_tiled(6), _tiled(6)],
        out_shape=[jax.ShapeDtypeStruct((npad, 6), f32),
                   jax.ShapeDtypeStruct((npad, 6), f32)],
    )(idx, ew, xg, xg, p['gc_rel_W'], b2('gc_rel'), p['gc_root_W'],
      p['nn2_1_W'], b2('nn2_1'), p['nn2_2_W'], b2('nn2_2'),
      p['nn2_3_W'], b2('nn2_3'), p['nn3_1_W'], b2('nn3_1'),
      p['nn3_2_W'], b2('nn3_2'), p['nn3_3_W'], b2('nn3_3'))

    return (ids[:n], p4[:n], ygen_id, ygen)


# speedup vs baseline: 5.6037x; 1.3047x over previous
import functools

import jax
import jax.numpy as jnp
from jax import lax
from jax.experimental import pallas as pl
from jax.experimental.pallas import tpu as pltpu, tpu_sc as plsc

_K = 16
_R = 256


def _lrelu(t):
    return jnp.where(t >= 0, t, 0.01 * t)


def _dot(a, b):
    return jnp.dot(a, b, preferred_element_type=jnp.float32)


def _encoder_body(x_ref, w1, b1, w2, b2, w3, b3, ws, bs, wh, bh,
                  x1_ref, s_ref, hp_ref):
    x = x_ref[...]
    h = _lrelu(_dot(x, w1[...]) + b1[...])
    h = _lrelu(_dot(h, w2[...]) + b2[...])
    x1 = _dot(h, w3[...]) + b3[...]
    x1_ref[...] = x1
    s_ref[...] = _dot(x1, ws[...]) + bs[...]
    hp_ref[...] = _dot(x1, wh[...]) + bh[...]


def _knn_body(n_real, s_ref, sT_ref, idx_ref):
    tile = pl.program_id(0)
    R = s_ref.shape[0]
    npad = sT_ref.shape[1]
    s_r = s_ref[...]
    sT = sT_ref[...]
    sq_r = jnp.sum(s_r * s_r, axis=1, keepdims=True)
    sq_c = jnp.sum(sT * sT, axis=0, keepdims=True)
    d2 = sq_r + sq_c - 2.0 * _dot(s_r, sT[:4, :])
    col = jax.lax.broadcasted_iota(jnp.int32, (R, npad), 1)
    row = tile * R + jax.lax.broadcasted_iota(jnp.int32, (R, npad), 0)
    big = jnp.float32(1e30)
    d2 = jnp.where((col >= n_real) | (col == row), big, d2)

    idx_cols = []
    for _ in range(_K):
        i_sel = jnp.argmin(d2, axis=1, keepdims=True).astype(jnp.int32)
        idx_cols.append(i_sel)
        d2 = jnp.where(col == i_sel, big, d2)
    idx_ref[...] = jnp.concatenate(idx_cols, axis=1)


def _agg1_body(ghs_ref, s_ref, x1_ref, wo, bo, xg_ref, ew_ref):
    ghs = ghs_ref[...]
    s_r = s_ref[...]
    big = jnp.float32(1e30)
    mean_acc = jnp.zeros((ghs.shape[0], 22), jnp.float32)
    max_acc = jnp.full((ghs.shape[0], 22), -big, jnp.float32)
    ew_cols = []
    for k in range(_K):
        blk = ghs[:, k * 128:k * 128 + 26]
        diff = blk[:, 22:26] - s_r
        d2e = jnp.sum(diff * diff, axis=1, keepdims=True)
        w_t = jnp.exp(-10.0 * d2e)
        msg = blk[:, :22] * w_t
        mean_acc = mean_acc + msg
        max_acc = jnp.maximum(max_acc, msg)
        ew_cols.append(w_t)
    mean_acc = mean_acc * (1.0 / _K)
    out = jnp.concatenate([mean_acc, max_acc, x1_ref[...]], axis=1)
    xg_ref[...] = _lrelu(_dot(out, wo[...]) + bo[...])
    ew_ref[...] = jnp.concatenate(ew_cols, axis=1)


def _sc_gather(npad, table_hbm, idx_hbm, out_hbm, idx_v, rows_v, sem):
    info = plsc.get_sparse_core_info()
    nc, ns = info.num_cores, info.num_subcores
    nw = nc * ns
    b = npad * _K
    b_per_w = b // nw
    chunk = idx_v.shape[0]
    wid = lax.axis_index("s") * nc + lax.axis_index("c")
    for c in range(b_per_w // chunk):
        off = wid * b_per_w + c * chunk
        pltpu.sync_copy(idx_hbm.at[pl.ds(off, chunk)], idx_v)
        pltpu.async_copy(table_hbm.at[idx_v], rows_v, sem).wait()
        pltpu.sync_copy(rows_v, out_hbm.at[pl.ds(off, chunk)])


def _head_body(g_ref, ew_ref, xg_ref, wrel, brel, wroot,
               w21, b21, w22, b22, w23, b23, w31, b31, w32, b32, w33, b33,
               ids_ref, p4_ref):
    g = g_ref[...]
    ew = ew_ref[...]
    agg2 = jnp.zeros((g.shape[0], 32), jnp.float32)
    for k in range(_K):
        agg2 = agg2 + g[:, k * 128:k * 128 + 32] * ew[:, k:k + 1]
    xg = xg_ref[...]
    xc = _lrelu(_dot(agg2, wrel[...]) + brel[...] + _dot(xg, wroot[...]))
    h2 = _lrelu(_dot(xc, w21[...]) + b21[...])
    h2 = _lrelu(_dot(h2, w22[...]) + b22[...])
    ids = _dot(h2, w23[...]) + b23[...]
    z = jnp.concatenate([xc, ids], axis=1)
    h3 = _lrelu(_dot(z, w31[...]) + b31[...])
    h3 = _lrelu(_dot(h3, w32[...]) + b32[...])
    p4 = _dot(h3, w33[...]) + b33[...]
    ids_ref[...] = ids
    p4_ref[...] = p4


def _full(shape):
    nd = len(shape)
    return pl.BlockSpec(shape, lambda i: (0,) * nd)


def _tiled(c, r=_R):
    return pl.BlockSpec((r, c), lambda i: (i, 0))


def kernel(x, ygen_id, ygen, params):
    p = params
    n = x.shape[0]
    npad = -(-n // _R) * _R
    grid = npad // _R
    f32 = jnp.float32

    xp = jnp.zeros((npad, 12), f32).at[:n, :].set(x)

    def b2(name):
        return p[name + '_b'].reshape(1, -1)

    x1, s, hp = pl.pallas_call(
        _encoder_body,
        grid=(grid,),
        in_specs=[_tiled(12)] + [_full(p[k].shape if k.endswith('_W')
                                       else (1, p[k[:-2] + '_b'].shape[0]))
                                 for k in ['nn1_1_W', 'nn1_1_b', 'nn1_2_W',
                                           'nn1_2_b', 'nn1_3_W', 'nn1_3_b',
                                           'gn_s_W', 'gn_s_b', 'gn_h_W',
                                           'gn_h_b']],
        out_specs=[_tiled(12), _tiled(4), _tiled(22)],
        out_shape=[jax.ShapeDtypeStruct((npad, 12), f32),
                   jax.ShapeDtypeStruct((npad, 4), f32),
                   jax.ShapeDtypeStruct((npad, 22), f32)],
    )(xp, p['nn1_1_W'], b2('nn1_1'), p['nn1_2_W'], b2('nn1_2'),
      p['nn1_3_W'], b2('nn1_3'), p['gn_s_W'], b2('gn_s'),
      p['gn_h_W'], b2('gn_h'))

    sT = jnp.zeros((8, npad), f32).at[:4, :].set(s.T)
    hsp = jnp.concatenate([hp, s, jnp.zeros((npad, 102), f32)], axis=1)

    idx = pl.pallas_call(
        functools.partial(_knn_body, n),
        grid=(grid,),
        in_specs=[_tiled(4), _full((8, npad))],
        out_specs=_tiled(_K),
        out_shape=jax.ShapeDtypeStruct((npad, _K), jnp.int32),
    )(s, sT)

    bt = npad * _K
    chunk = 640
    mesh = plsc.VectorSubcoreMesh(core_axis_name="c", subcore_axis_name="s")
    idx_flat = idx.reshape(-1)

    def sc_gather(table):
        return pl.kernel(
            functools.partial(_sc_gather, npad),
            mesh=mesh,
            out_type=jax.ShapeDtypeStruct((bt, 128), f32),
            scratch_types=[pltpu.VMEM((chunk,), jnp.int32),
                           pltpu.VMEM((chunk, 128), f32),
                           pltpu.SemaphoreType.DMA],
        )(table, idx_flat)

    ghs = sc_gather(hsp).reshape(npad, _K * 128)

    xg, ew = pl.pallas_call(
        _agg1_body,
        grid=(grid,),
        in_specs=[_tiled(_K * 128), _tiled(4), _tiled(12),
                  _full((56, 32)), _full((1, 32))],
        out_specs=[_tiled(32), _tiled(_K)],
        out_shape=[jax.ShapeDtypeStruct((npad, 32), f32),
                   jax.ShapeDtypeStruct((npad, _K), f32)],
    )(ghs, s, x1, p['gn_o_W'], b2('gn_o'))

    xgp = jnp.zeros((npad, 128), f32).at[:, :32].set(xg)
    gathered = sc_gather(xgp)

    ids, p4 = pl.pallas_call(
        _head_body,
        grid=(grid,),
        in_specs=[_tiled(_K * 128), _tiled(_K), _tiled(32),
                  _full((32, 32)), _full((1, 32)), _full((32, 32)),
                  _full((32, 125)), _full((1, 125)),
                  _full((125, 125)), _full((1, 125)),
                  _full((125, 6)), _full((1, 6)),
                  _full((38, 125)), _full((1, 125)),
                  _full((125, 125)), _full((1, 125)),
                  _full((125, 6)), _full((1, 6))],
        out_specs=[_tiled(6), _tiled(6)],
        out_shape=[jax.ShapeDtypeStruct((npad, 6), f32),
                   jax.ShapeDtypeStruct((npad, 6), f32)],
    )(gathered.reshape(npad, _K * 128), ew, xg,
      p['gc_rel_W'], b2('gc_rel'), p['gc_root_W'],
      p['nn2_1_W'], b2('nn2_1'), p['nn2_2_W'], b2('nn2_2'),
      p['nn2_3_W'], b2('nn2_3'), p['nn3_1_W'], b2('nn3_1'),
      p['nn3_2_W'], b2('nn3_2'), p['nn3_3_W'], b2('nn3_3'))

    return (ids[:n], p4[:n], ygen_id, ygen)
